# Initial kernel scaffold; baseline (speedup 1.0000x reference)
import jax
import jax.numpy as jnp
from jax.experimental import pallas as pl


def kernel(x, emb1, emb2, attention_W, out_W, out_b):
    raise NotImplementedError("write your pallas kernel here")


# speedup vs baseline: 2.5120x; 2.5120x over previous
import functools

import jax
import jax.numpy as jnp
from jax import lax
from jax.experimental import pallas as pl
from jax.experimental.pallas import tpu as pltpu
from jax.experimental.pallas import tpu_sc as plsc

NC, NS = 2, 16
NW = NC * NS
NB = 4
L = 16


def _afm_sc(idx2, idx1, t2, t1, wt, w0b, B, F, D):
    GL = NB * F
    nblk = B // (NB * NW)
    mesh = plsc.VectorSubcoreMesh(core_axis_name="c", subcore_axis_name="s")

    @functools.partial(
        pl.kernel,
        out_type=jax.ShapeDtypeStruct((B,), jnp.float32),
        mesh=mesh,
        compiler_params=pltpu.CompilerParams(
            needs_layout_passes=False, use_tc_tiling_on_sc=False),
        scratch_types=[
            pltpu.VMEM((F, GL), jnp.int32),
            pltpu.VMEM((GL,), jnp.int32),
            pltpu.VMEM((GL,), jnp.float32),
            pltpu.VMEM((F, GL, D), jnp.float32),
            pltpu.VMEM((2 * NB, L), jnp.float32),
            pltpu.VMEM((B // NW,), jnp.float32),
            pltpu.VMEM((D,), jnp.float32),
            pltpu.VMEM((2, L), jnp.float32),
            pltpu.SemaphoreType.DMA,
        ],
    )
    def k(idx2_hbm, idx1_hbm, t2_hbm, t1_hbm, wt_hbm, w0b_hbm, out_hbm,
          idx2_v, idx1_v, e1_v, rows_v, acc_v, out_v, wt_v, w0b_v, sem):
        wid = lax.axis_index("s") * NC + lax.axis_index("c")
        pltpu.sync_copy(wt_hbm, wt_v)
        pltpu.sync_copy(w0b_hbm, w0b_v)
        wt0 = wt_v[pl.ds(0, L)]
        wt1 = wt_v[pl.ds(L, L)]
        w0v = w0b_v[0]
        bias = w0b_v[1]
        jmask = lax.iota(jnp.int32, L) >= (2 * L - F)
        lane0 = lax.iota(jnp.int32, L) == 0
        zero = jnp.zeros((L,), jnp.float32)

        def block(blk_local, carry):
            blk = wid * nblk + blk_local
            pltpu.sync_copy(idx2_hbm.at[blk], idx2_v)
            pltpu.sync_copy(idx1_hbm.at[pl.ds(blk * GL, GL)], idx1_v)
            handles = []
            for i in range(F):
                handles.append(
                    pltpu.async_copy(t2_hbm.at[idx2_v.at[i]], rows_v.at[i], sem))
            handles.append(pltpu.async_copy(t1_hbm.at[idx1_v], e1_v, sem))
            for r in range(2 * NB):
                acc_v[r] = zero
            for h in handles:
                h.wait()

            def comp(i, c):
                for bb in range(NB):
                    base = bb * F
                    s0 = rows_v[i, base, pl.ds(0, L)]
                    s1 = rows_v[i, base, pl.ds(L, L)]
                    for j in range(1, F):
                        s0 = s0 + rows_v[i, base + j, pl.ds(0, L)]
                        s1 = s1 + rows_v[i, base + j, pl.ds(L, L)]
                    d0 = rows_v[i, base + i, pl.ds(0, L)]
                    d1 = rows_v[i, base + i, pl.ds(L, L)]
                    plsc.addupdate(acc_v.at[2 * bb], d0 * s0)
                    plsc.addupdate(acc_v.at[2 * bb + 1], d1 * s1)
                return c

            lax.fori_loop(0, F, comp, 0)

            for bb in range(NB):
                a0 = acc_v[2 * bb]
                a1 = acc_v[2 * bb + 1]
                e0 = e1_v[pl.ds(bb * F, L)]
                eb = e1_v[pl.ds(bb * F + F - L, L)]
                eb = jnp.where(jmask, eb, 0.0)
                tvec = a0 * wt0 + a1 * wt1 + bias + w0v * (e0 + eb)
                tot = jnp.broadcast_to(jnp.sum(tvec), (L,))
                pos = jnp.broadcast_to(blk_local * NB + bb, (L,))
                plsc.store_scatter(out_v, [pos], tot, mask=lane0)
            return carry

        lax.fori_loop(0, nblk, block, 0)
        pltpu.sync_copy(out_v, out_hbm.at[pl.ds(wid * (B // NW), B // NW)])

    return k(idx2, idx1, t2, t1, wt, w0b)


def kernel(x, emb1, emb2, attention_W, out_W, out_b):
    B = x.shape[0]
    F, V, D = emb2.shape
    x32 = x.astype(jnp.int32)
    foff = jnp.arange(F, dtype=jnp.int32) * V
    xr = x32.reshape(B // NB, NB, F)
    idx2 = (foff[None, :, None, None] + xr[:, None, :, :]).reshape(B // NB, F, NB * F)
    idx1 = (x32 + foff[None, :]).reshape(B * F)
    t2 = emb2.reshape(F * V, D)
    t1 = emb1.reshape(F * V)
    wt = out_W[1:, 0].astype(jnp.float32)
    w0b = jnp.stack([
        jnp.full((L,), out_W[0, 0], jnp.float32),
        jnp.full((L,), out_b[0] / L, jnp.float32),
    ])
    out = _afm_sc(idx2, idx1, t2, t1, wt, w0b, B, F, D)
    return out.reshape(B, 1)


# speedup vs baseline: 2.5425x; 1.0121x over previous
import functools

import jax
import jax.numpy as jnp
from jax import lax
from jax.experimental import pallas as pl
from jax.experimental.pallas import tpu as pltpu
from jax.experimental.pallas import tpu_sc as plsc

NC, NS = 2, 16
NW = NC * NS
NB = 4
L = 16


def _afm_sc(xi, idx1, emb2, t1, wt, w0b, B, F, D):
    GL = NB * F
    nblk = B // (NB * NW)
    mesh = plsc.VectorSubcoreMesh(core_axis_name="c", subcore_axis_name="s")

    @functools.partial(
        pl.kernel,
        out_type=jax.ShapeDtypeStruct((B,), jnp.float32),
        mesh=mesh,
        compiler_params=pltpu.CompilerParams(
            needs_layout_passes=False, use_tc_tiling_on_sc=False),
        scratch_types=[
            pltpu.VMEM((GL,), jnp.int32),
            pltpu.VMEM((GL,), jnp.int32),
            pltpu.VMEM((GL,), jnp.float32),
            pltpu.VMEM((F, GL, D), jnp.float32),
            pltpu.VMEM((2 * NB, L), jnp.float32),
            pltpu.VMEM((B // NW,), jnp.float32),
            pltpu.VMEM((D,), jnp.float32),
            pltpu.VMEM((2, L), jnp.float32),
            pltpu.SemaphoreType.DMA,
        ],
    )
    def k(xi_hbm, idx1_hbm, t2_hbm, t1_hbm, wt_hbm, w0b_hbm, out_hbm,
          xi_v, idx1_v, e1_v, rows_v, acc_v, out_v, wt_v, w0b_v, sem):
        wid = lax.axis_index("s") * NC + lax.axis_index("c")
        pltpu.sync_copy(wt_hbm, wt_v)
        pltpu.sync_copy(w0b_hbm, w0b_v)
        wt0 = wt_v[pl.ds(0, L)]
        wt1 = wt_v[pl.ds(L, L)]
        w0v = w0b_v[0]
        bias = w0b_v[1]
        jmask = lax.iota(jnp.int32, L) >= (2 * L - F)
        lane0 = lax.iota(jnp.int32, L) == 0
        zero = jnp.zeros((L,), jnp.float32)

        def block(blk_local, carry):
            blk = wid * nblk + blk_local
            pltpu.sync_copy(xi_hbm.at[blk], xi_v)
            pltpu.sync_copy(idx1_hbm.at[pl.ds(blk * GL, GL)], idx1_v)
            handles = []
            for i in range(F):
                handles.append(
                    pltpu.async_copy(t2_hbm.at[i].at[xi_v], rows_v.at[i], sem))
            handles.append(pltpu.async_copy(t1_hbm.at[idx1_v], e1_v, sem))
            for r in range(2 * NB):
                acc_v[r] = zero
            for h in handles:
                h.wait()

            def comp(i, c):
                for bb in range(NB):
                    base = bb * F
                    s0 = rows_v[i, base, pl.ds(0, L)]
                    s1 = rows_v[i, base, pl.ds(L, L)]
                    for j in range(1, F):
                        s0 = s0 + rows_v[i, base + j, pl.ds(0, L)]
                        s1 = s1 + rows_v[i, base + j, pl.ds(L, L)]
                    d0 = rows_v[i, base + i, pl.ds(0, L)]
                    d1 = rows_v[i, base + i, pl.ds(L, L)]
                    plsc.addupdate(acc_v.at[2 * bb], d0 * s0)
                    plsc.addupdate(acc_v.at[2 * bb + 1], d1 * s1)
                return c

            lax.fori_loop(0, F, comp, 0)

            for bb in range(NB):
                a0 = acc_v[2 * bb]
                a1 = acc_v[2 * bb + 1]
                e0 = e1_v[pl.ds(bb * F, L)]
                eb = e1_v[pl.ds(bb * F + F - L, L)]
                eb = jnp.where(jmask, eb, 0.0)
                tvec = a0 * wt0 + a1 * wt1 + bias + w0v * (e0 + eb)
                tot = jnp.broadcast_to(jnp.sum(tvec), (L,))
                pos = jnp.broadcast_to(blk_local * NB + bb, (L,))
                plsc.store_scatter(out_v, [pos], tot, mask=lane0)
            return carry

        lax.fori_loop(0, nblk, block, 0)
        pltpu.sync_copy(out_v, out_hbm.at[pl.ds(wid * (B // NW), B // NW)])

    return k(xi, idx1, emb2, t1, wt, w0b)


def kernel(x, emb1, emb2, attention_W, out_W, out_b):
    B = x.shape[0]
    F, V, D = emb2.shape
    x32 = x.astype(jnp.int32)
    foff = jnp.arange(F, dtype=jnp.int32) * V
    xi = x32.reshape(B // NB, NB * F)
    idx1 = (x32 + foff[None, :]).reshape(B * F)
    t1 = emb1.reshape(F * V)
    wt = out_W[1:, 0].astype(jnp.float32)
    w0b = jnp.stack([
        jnp.full((L,), out_W[0, 0], jnp.float32),
        jnp.full((L,), out_b[0] / L, jnp.float32),
    ])
    out = _afm_sc(xi, idx1, emb2, t1, wt, w0b, B, F, D)
    return out.reshape(B, 1)
